# baseline (speedup 1.0000x reference)
import functools

import jax
import jax.numpy as jnp
import numpy as np
from jax.experimental import pallas as pl

_TOPK = 10
_ENTROPY_WEIGHT = 2.0
_EPS = 1e-08
_BLK = 256
_N_PROB = 100


def _scat_const() -> np.ndarray:
    s = np.zeros((_N_PROB, 128 * _TOPK), np.float32)
    for u in range(_TOPK):
        for v in range(_TOPK):
            for k in range(_TOPK):
                s[10 * u + v, 128 * u + 10 * v + k] = 1.0
    return s


def _rsum_const() -> np.ndarray:
    r = np.zeros((_N_PROB, _TOPK), np.float32)
    for j in range(_N_PROB):
        r[j, j % 10] = 1.0
    return r


def _softmax_entropy_kernel(p_ref, q_ref, ent_ref):
    p = p_ref[...]
    m = jnp.max(p, axis=1, keepdims=True)
    e = jnp.exp(p - m)
    q = e / jnp.sum(e, axis=1, keepdims=True)
    q_ref[...] = q
    col_mean = jnp.mean(q, axis=0)
    x = jnp.clip(col_mean, _EPS, None)
    ent_ref[...] = jnp.reshape(-jnp.sum(x * jnp.log(x)), (1, 1))


def _knn_loss_kernel(a_blk_ref, a_t_ref, q_ref, scat_ref, rsum_ref, acc_ref):
    i = pl.program_id(0)
    feats = jax.lax.dot_general(
        a_blk_ref[...].astype(jnp.bfloat16),
        a_t_ref[...].astype(jnp.bfloat16),
        (((1,), (0,)), ((), ())),
        preferred_element_type=jnp.float32,
    )
    blk, n_rows = feats.shape
    q_bf = q_ref[...].astype(jnp.bfloat16)
    qb = q_ref[pl.ds(i * blk, blk), :]
    h = jax.lax.dot_general(
        qb, scat_ref[...],
        (((1,), (0,)), ((), ())),
        precision=jax.lax.Precision.HIGHEST,
        preferred_element_type=jnp.float32,
    )
    p_sum = jnp.zeros((blk, _N_PROB), jnp.float32)
    neg_inf = jnp.float32(-jnp.inf)
    m = jnp.max(feats, axis=1, keepdims=True)
    for t in range(_TOPK + 1):
        if t > 0:
            onehot = feats == m
            g = jax.lax.dot_general(
                onehot.astype(jnp.bfloat16), q_bf,
                (((1,), (0,)), ((), ())),
                preferred_element_type=jnp.float32,
            )
            u = t - 1
            p_sum = p_sum + h[:, 128 * u:128 * u + _N_PROB] * g
        if t < _TOPK:
            m = jnp.max(
                jnp.where(feats < m, feats, neg_inf), axis=1, keepdims=True
            )
    sim = jax.lax.dot_general(
        p_sum, rsum_ref[...],
        (((1,), (0,)), ((), ())),
        precision=jax.lax.Precision.HIGHEST,
        preferred_element_type=jnp.float32,
    )
    log_sim = jnp.clip(jnp.log(sim), -100.0, None)

    @pl.when(i == 0)
    def _init():
        acc_ref[...] = jnp.zeros((1, 1), jnp.float32)

    acc_ref[...] += jnp.reshape(jnp.sum(log_sim), (1, 1))


@functools.partial(jax.jit, static_argnames=())
def kernel(anchors, prob):
    b, c, h, w = anchors.shape
    n_rows = b * h * w
    a = jnp.transpose(anchors, (0, 3, 2, 1)).reshape(n_rows, c)
    p = jnp.transpose(prob, (0, 3, 2, 1)).reshape(n_rows, -1)

    q, ent = pl.pallas_call(
        _softmax_entropy_kernel,
        out_shape=(
            jax.ShapeDtypeStruct((n_rows, p.shape[1]), jnp.float32),
            jax.ShapeDtypeStruct((1, 1), jnp.float32),
        ),
    )(p)

    num_blocks = n_rows // _BLK
    acc = pl.pallas_call(
        _knn_loss_kernel,
        grid=(num_blocks,),
        in_specs=[
            pl.BlockSpec((_BLK, c), lambda i: (i, 0)),
            pl.BlockSpec((c, n_rows), lambda i: (0, 0)),
            pl.BlockSpec((n_rows, _N_PROB), lambda i: (0, 0)),
            pl.BlockSpec((_N_PROB, 128 * _TOPK), lambda i: (0, 0)),
            pl.BlockSpec((_N_PROB, _TOPK), lambda i: (0, 0)),
        ],
        out_specs=pl.BlockSpec((1, 1), lambda i: (0, 0)),
        out_shape=jax.ShapeDtypeStruct((1, 1), jnp.float32),
    )(a, a.T, q, _scat_const(), _rsum_const())

    consistency = -acc[0, 0] / (n_rows * _TOPK)
    entropy = ent[0, 0]
    total = consistency - _ENTROPY_WEIGHT * entropy
    return (total, consistency, entropy)


# speedup vs baseline: 1.1166x; 1.1166x over previous
import functools

import jax
import jax.numpy as jnp
import numpy as np
from jax.experimental import pallas as pl

_TOPK = 10
_ENTROPY_WEIGHT = 2.0
_EPS = 1e-08
_BLK = 256
_N_PROB = 100


def _scat_const() -> np.ndarray:
    s = np.zeros((_N_PROB, 128 * _TOPK), np.float32)
    for u in range(_TOPK):
        for v in range(_TOPK):
            for k in range(_TOPK):
                s[10 * u + v, 128 * u + 10 * v + k] = 1.0
    return s


def _rsum_const() -> np.ndarray:
    r = np.zeros((_N_PROB, _TOPK), np.float32)
    for j in range(_N_PROB):
        r[j, j % 10] = 1.0
    return r


def _softmax_entropy_kernel(p_ref, q_ref, ent_ref):
    p = p_ref[...]
    m = jnp.max(p, axis=1, keepdims=True)
    e = jnp.exp(p - m)
    q = e / jnp.sum(e, axis=1, keepdims=True)
    q_ref[...] = q
    col_mean = jnp.mean(q, axis=0)
    x = jnp.clip(col_mean, _EPS, None)
    ent_ref[...] = jnp.reshape(-jnp.sum(x * jnp.log(x)), (1, 1))


def _knn_loss_kernel(a_blk_ref, a_t_ref, q_ref, scat_ref, rsum_ref, acc_ref):
    i = pl.program_id(0)
    feats = jax.lax.dot_general(
        a_blk_ref[...].astype(jnp.bfloat16),
        a_t_ref[...].astype(jnp.bfloat16),
        (((1,), (0,)), ((), ())),
        preferred_element_type=jnp.float32,
    )
    blk, n_rows = feats.shape
    q_bf = q_ref[...].astype(jnp.bfloat16)
    qb = q_ref[pl.ds(i * blk, blk), :]
    h = jax.lax.dot_general(
        qb, scat_ref[...],
        (((1,), (0,)), ((), ())),
        preferred_element_type=jnp.float32,
    )
    p_sum = jnp.zeros((blk, _N_PROB), jnp.float32)
    work = feats
    for t in range(_TOPK + 1):
        m = jnp.max(work, axis=1, keepdims=True)
        onehot = work == m
        if t > 0:
            g = jax.lax.dot_general(
                onehot.astype(jnp.bfloat16), q_bf,
                (((1,), (0,)), ((), ())),
                preferred_element_type=jnp.float32,
            )
            u = t - 1
            p_sum = p_sum + h[:, 128 * u:128 * u + _N_PROB] * g
        if t < _TOPK:
            work = jnp.where(onehot, -jnp.inf, work)
    sim = jax.lax.dot_general(
        p_sum, rsum_ref[...],
        (((1,), (0,)), ((), ())),
        precision=jax.lax.Precision.HIGHEST,
        preferred_element_type=jnp.float32,
    )
    log_sim = jnp.clip(jnp.log(sim), -100.0, None)

    @pl.when(i == 0)
    def _init():
        acc_ref[...] = jnp.zeros((1, 1), jnp.float32)

    acc_ref[...] += jnp.reshape(jnp.sum(log_sim), (1, 1))


@functools.partial(jax.jit, static_argnames=())
def kernel(anchors, prob):
    b, c, h, w = anchors.shape
    n_rows = b * h * w
    a = jnp.transpose(anchors, (0, 3, 2, 1)).reshape(n_rows, c)
    p = jnp.transpose(prob, (0, 3, 2, 1)).reshape(n_rows, -1)

    q, ent = pl.pallas_call(
        _softmax_entropy_kernel,
        out_shape=(
            jax.ShapeDtypeStruct((n_rows, p.shape[1]), jnp.float32),
            jax.ShapeDtypeStruct((1, 1), jnp.float32),
        ),
    )(p)

    num_blocks = n_rows // _BLK
    acc = pl.pallas_call(
        _knn_loss_kernel,
        grid=(num_blocks,),
        in_specs=[
            pl.BlockSpec((_BLK, c), lambda i: (i, 0)),
            pl.BlockSpec((c, n_rows), lambda i: (0, 0)),
            pl.BlockSpec((n_rows, _N_PROB), lambda i: (0, 0)),
            pl.BlockSpec((_N_PROB, 128 * _TOPK), lambda i: (0, 0)),
            pl.BlockSpec((_N_PROB, _TOPK), lambda i: (0, 0)),
        ],
        out_specs=pl.BlockSpec((1, 1), lambda i: (0, 0)),
        out_shape=jax.ShapeDtypeStruct((1, 1), jnp.float32),
    )(a, a.T, q, _scat_const(), _rsum_const())

    consistency = -acc[0, 0] / (n_rows * _TOPK)
    entropy = ent[0, 0]
    total = consistency - _ENTROPY_WEIGHT * entropy
    return (total, consistency, entropy)


# speedup vs baseline: 1.1389x; 1.0200x over previous
import functools

import jax
import jax.numpy as jnp
import numpy as np
from jax.experimental import pallas as pl
from jax.experimental.pallas import tpu as pltpu

_TOPK = 10
_ENTROPY_WEIGHT = 2.0
_EPS = 1e-08
_BLK = 256
_N_PROB = 100


def _scat_const() -> np.ndarray:
    s = np.zeros((_N_PROB, 128 * _TOPK), np.float32)
    for u in range(_TOPK):
        for v in range(_TOPK):
            for k in range(_TOPK):
                s[10 * u + v, 128 * u + 10 * v + k] = 1.0
    return s


def _rsum_const() -> np.ndarray:
    r = np.zeros((_N_PROB, _TOPK), np.float32)
    for j in range(_N_PROB):
        r[j, j % 10] = 1.0
    return r


def _knn_loss_kernel(a_ref, p_ref, scat_ref, rsum_ref, acc_ref, ent_ref,
                     q_ref, qbf_ref):
    i = pl.program_id(0)

    @pl.when(i == 0)
    def _softmax_entropy():
        p = p_ref[...]
        mx = jnp.max(p, axis=1, keepdims=True)
        e = jnp.exp(p - mx)
        q = e / jnp.sum(e, axis=1, keepdims=True)
        q_ref[...] = q
        qbf_ref[...] = q.astype(jnp.bfloat16)
        col_mean = jnp.mean(q, axis=0)
        x = jnp.clip(col_mean, _EPS, None)
        ent_ref[...] = jnp.reshape(-jnp.sum(x * jnp.log(x)), (1, 1))
        acc_ref[...] = jnp.zeros((1, 1), jnp.float32)

    a_blk = a_ref[pl.ds(i * _BLK, _BLK), :].astype(jnp.bfloat16)
    feats = jax.lax.dot_general(
        a_blk, a_ref[...].astype(jnp.bfloat16),
        (((1,), (1,)), ((), ())),
        preferred_element_type=jnp.float32,
    )
    blk = _BLK
    q_bf = qbf_ref[...]
    qb = q_ref[pl.ds(i * blk, blk), :]
    h = jax.lax.dot_general(
        qb, scat_ref[...],
        (((1,), (0,)), ((), ())),
        preferred_element_type=jnp.float32,
    )
    p_sum = jnp.zeros((blk, _N_PROB), jnp.float32)
    work = feats
    for t in range(_TOPK + 1):
        m = jnp.max(work, axis=1, keepdims=True)
        onehot = work == m
        if t > 0:
            g = jax.lax.dot_general(
                onehot.astype(jnp.bfloat16), q_bf,
                (((1,), (0,)), ((), ())),
                preferred_element_type=jnp.float32,
            )
            u = t - 1
            p_sum = p_sum + h[:, 128 * u:128 * u + _N_PROB] * g
        if t < _TOPK:
            work = jnp.where(onehot, -jnp.inf, work)
    sim = jax.lax.dot_general(
        p_sum, rsum_ref[...],
        (((1,), (0,)), ((), ())),
        precision=jax.lax.Precision.HIGHEST,
        preferred_element_type=jnp.float32,
    )
    log_sim = jnp.clip(jnp.log(sim), -100.0, None)
    acc_ref[...] += jnp.reshape(jnp.sum(log_sim), (1, 1))


@functools.partial(jax.jit, static_argnames=())
def kernel(anchors, prob):
    b, c, h, w = anchors.shape
    n_rows = b * h * w
    a = jnp.transpose(anchors, (0, 3, 2, 1)).reshape(n_rows, c)
    p = jnp.transpose(prob, (0, 3, 2, 1)).reshape(n_rows, -1)

    num_blocks = n_rows // _BLK
    acc, ent = pl.pallas_call(
        _knn_loss_kernel,
        grid=(num_blocks,),
        in_specs=[
            pl.BlockSpec((n_rows, c), lambda i: (0, 0)),
            pl.BlockSpec((n_rows, _N_PROB), lambda i: (0, 0)),
            pl.BlockSpec((_N_PROB, 128 * _TOPK), lambda i: (0, 0)),
            pl.BlockSpec((_N_PROB, _TOPK), lambda i: (0, 0)),
        ],
        out_specs=(
            pl.BlockSpec((1, 1), lambda i: (0, 0)),
            pl.BlockSpec((1, 1), lambda i: (0, 0)),
        ),
        out_shape=(
            jax.ShapeDtypeStruct((1, 1), jnp.float32),
            jax.ShapeDtypeStruct((1, 1), jnp.float32),
        ),
        scratch_shapes=[
            pltpu.VMEM((n_rows, _N_PROB), jnp.float32),
            pltpu.VMEM((n_rows, _N_PROB), jnp.bfloat16),
        ],
    )(a, p, _scat_const(), _rsum_const())

    consistency = -acc[0, 0] / (n_rows * _TOPK)
    entropy = ent[0, 0]
    total = consistency - _ENTROPY_WEIGHT * entropy
    return (total, consistency, entropy)


# speedup vs baseline: 1.2198x; 1.0710x over previous
import functools

import jax
import jax.numpy as jnp
import numpy as np
from jax.experimental import pallas as pl
from jax.experimental.pallas import tpu as pltpu

_TOPK = 10
_ENTROPY_WEIGHT = 2.0
_EPS = 1e-08
_BLK = 256
_N_PROB = 100


def _scat_const() -> np.ndarray:
    s = np.zeros((_N_PROB, 128 * _TOPK), np.float32)
    for u in range(_TOPK):
        for v in range(_TOPK):
            for k in range(_TOPK):
                s[10 * u + v, 128 * u + 10 * v + k] = 1.0
    return s


def _rsum_const() -> np.ndarray:
    r = np.zeros((_N_PROB, _TOPK), np.float32)
    for j in range(_N_PROB):
        r[j, j % 10] = 1.0
    return r


def _knn_loss_kernel(a_ref, p_ref, scat_ref, rsum_ref, acc_ref, ent_ref,
                     q_ref, qbf_ref):
    i = pl.program_id(0)

    @pl.when(i == 0)
    def _softmax_entropy():
        p = p_ref[...]
        mx = jnp.max(p, axis=1, keepdims=True)
        e = jnp.exp(p - mx)
        q = e / jnp.sum(e, axis=1, keepdims=True)
        q_ref[...] = q
        qbf_ref[...] = q.astype(jnp.bfloat16)
        col_mean = jnp.mean(q, axis=0)
        x = jnp.clip(col_mean, _EPS, None)
        ent_ref[...] = jnp.reshape(-jnp.sum(x * jnp.log(x)), (1, 1))
        acc_ref[...] = jnp.zeros((1, 1), jnp.float32)

    a_blk = a_ref[pl.ds(i * _BLK, _BLK), :].astype(jnp.bfloat16)
    feats = jax.lax.dot_general(
        a_blk, a_ref[...].astype(jnp.bfloat16),
        (((1,), (1,)), ((), ())),
        preferred_element_type=jnp.float32,
    )
    blk = _BLK
    q_bf = qbf_ref[...]
    qb = q_ref[pl.ds(i * blk, blk), :]
    h = jax.lax.dot_general(
        qb, scat_ref[...],
        (((1,), (0,)), ((), ())),
        preferred_element_type=jnp.float32,
    )
    p_sum = jnp.zeros((blk, _N_PROB), jnp.float32)
    work = feats
    for t in range(_TOPK + 1):
        m = jnp.max(work, axis=1, keepdims=True)
        onehot = work == m
        if t > 0:
            g = jax.lax.dot_general(
                onehot.astype(jnp.float32), q_bf,
                (((1,), (0,)), ((), ())),
                preferred_element_type=jnp.float32,
            )
            u = t - 1
            p_sum = p_sum + h[:, 128 * u:128 * u + _N_PROB] * g
        if t < _TOPK:
            work = jnp.where(onehot, -jnp.inf, work)
    sim = jax.lax.dot_general(
        p_sum, rsum_ref[...],
        (((1,), (0,)), ((), ())),
        precision=jax.lax.Precision.HIGHEST,
        preferred_element_type=jnp.float32,
    )
    log_sim = jnp.clip(jnp.log(sim), -100.0, None)
    acc_ref[...] += jnp.reshape(jnp.sum(log_sim), (1, 1))


@functools.partial(jax.jit, static_argnames=())
def kernel(anchors, prob):
    b, c, h, w = anchors.shape
    n_rows = b * h * w
    a = jnp.transpose(anchors, (0, 3, 2, 1)).reshape(n_rows, c)
    p = jnp.transpose(prob, (0, 3, 2, 1)).reshape(n_rows, -1)

    num_blocks = n_rows // _BLK
    acc, ent = pl.pallas_call(
        _knn_loss_kernel,
        grid=(num_blocks,),
        in_specs=[
            pl.BlockSpec((n_rows, c), lambda i: (0, 0)),
            pl.BlockSpec((n_rows, _N_PROB), lambda i: (0, 0)),
            pl.BlockSpec((_N_PROB, 128 * _TOPK), lambda i: (0, 0)),
            pl.BlockSpec((_N_PROB, _TOPK), lambda i: (0, 0)),
        ],
        out_specs=(
            pl.BlockSpec((1, 1), lambda i: (0, 0)),
            pl.BlockSpec((1, 1), lambda i: (0, 0)),
        ),
        out_shape=(
            jax.ShapeDtypeStruct((1, 1), jnp.float32),
            jax.ShapeDtypeStruct((1, 1), jnp.float32),
        ),
        scratch_shapes=[
            pltpu.VMEM((n_rows, _N_PROB), jnp.float32),
            pltpu.VMEM((n_rows, _N_PROB), jnp.bfloat16),
        ],
    )(a, p, _scat_const(), _rsum_const())

    consistency = -acc[0, 0] / (n_rows * _TOPK)
    entropy = ent[0, 0]
    total = consistency - _ENTROPY_WEIGHT * entropy
    return (total, consistency, entropy)
